# baseline (speedup 1.0000x reference)
import jax
import jax.numpy as jnp
from jax.experimental import pallas as pl
from jax.experimental.pallas import tpu as pltpu

_N = 10000
_NIN = 128
_NOUT = 127
_BC = 400
_NC = _N // _BC
_NBUF = 3


def _gcn_kernel(adj_hbm, x_ref, w_ref, b_ref, o_ref, bufs, xb_ref, in_sems):
    def start_in(c, slot):
        pltpu.make_async_copy(
            adj_hbm.at[pl.ds(c * _BC, _BC), :],
            bufs.at[slot],
            in_sems.at[slot],
        ).start()

    for s in range(_NBUF):
        start_in(s, s)

    xb_ref[...] = x_ref[...].astype(jnp.bfloat16)

    def step(c, carry):
        slot = jax.lax.rem(c, _NBUF)
        pltpu.make_async_copy(
            adj_hbm.at[pl.ds(c * _BC, _BC), :],
            bufs.at[slot],
            in_sems.at[slot],
        ).wait()
        a = bufs[slot].astype(jnp.bfloat16)
        acc = jnp.dot(a, xb_ref[...], preferred_element_type=jnp.float32)
        h = jnp.dot(
            acc.astype(jnp.bfloat16),
            w_ref[...],
            preferred_element_type=jnp.float32,
        )
        h = jnp.maximum(h + b_ref[...], 0.0)
        res = x_ref[pl.ds(c * _BC, _BC), :]
        o_ref[pl.ds(c * _BC, _BC), :] = (h + res)[:, :_NOUT]

        @pl.when(c + _NBUF < _NC)
        def _next():
            start_in(c + _NBUF, slot)

        return carry

    jax.lax.fori_loop(0, _NC, step, 0)


def kernel(x, adj, W, b):
    w_p = jnp.pad(W, ((0, 0), (0, _NIN - _NOUT))).astype(jnp.bfloat16)
    b_p = jnp.pad(b, (0, _NIN - _NOUT)).reshape(1, _NIN)
    return pl.pallas_call(
        _gcn_kernel,
        in_specs=[
            pl.BlockSpec(memory_space=pl.ANY),
            pl.BlockSpec(memory_space=pltpu.VMEM),
            pl.BlockSpec(memory_space=pltpu.VMEM),
            pl.BlockSpec(memory_space=pltpu.VMEM),
        ],
        out_specs=pl.BlockSpec(memory_space=pltpu.VMEM),
        out_shape=jax.ShapeDtypeStruct((_N, _NOUT), jnp.float32),
        scratch_shapes=[
            pltpu.VMEM((_NBUF, _BC, _N), jnp.float32),
            pltpu.VMEM((_N, _NIN), jnp.bfloat16),
            pltpu.SemaphoreType.DMA((_NBUF,)),
        ],
    )(adj, x, w_p, b_p)


# speedup vs baseline: 1.0373x; 1.0373x over previous
import jax
import jax.numpy as jnp
from jax.experimental import pallas as pl
from jax.experimental.pallas import tpu as pltpu

_N = 10000
_NIN = 128
_NOUT = 127
_BM = 400


def _gcn_kernel(adj_ref, x_ref, w_ref, b_ref, o_ref, xb_ref):
    i = pl.program_id(0)

    @pl.when(i == 0)
    def _cast_x_once():
        xb_ref[...] = x_ref[...].astype(jnp.bfloat16)

    a = adj_ref[...].astype(jnp.bfloat16)
    acc = jnp.dot(a, xb_ref[...], preferred_element_type=jnp.float32)
    h = jnp.dot(
        acc.astype(jnp.bfloat16),
        w_ref[...],
        preferred_element_type=jnp.float32,
    )
    h = jnp.maximum(h + b_ref[...], 0.0)
    res = x_ref[pl.ds(i * _BM, _BM), :]
    o_ref[...] = (h + res)[:, :_NOUT]


def kernel(x, adj, W, b):
    w_p = jnp.pad(W, ((0, 0), (0, _NIN - _NOUT))).astype(jnp.bfloat16)
    b_p = jnp.pad(b, (0, _NIN - _NOUT)).reshape(1, _NIN)
    return pl.pallas_call(
        _gcn_kernel,
        grid=(_N // _BM,),
        in_specs=[
            pl.BlockSpec((_BM, _N), lambda i: (i, 0)),
            pl.BlockSpec((_N, _NIN), lambda i: (0, 0)),
            pl.BlockSpec((_NIN, _NIN), lambda i: (0, 0)),
            pl.BlockSpec((1, _NIN), lambda i: (0, 0)),
        ],
        out_specs=pl.BlockSpec((_BM, _NOUT), lambda i: (i, 0)),
        out_shape=jax.ShapeDtypeStruct((_N, _NOUT), jnp.float32),
        scratch_shapes=[pltpu.VMEM((_N, _NIN), jnp.bfloat16)],
        compiler_params=pltpu.CompilerParams(
            dimension_semantics=("arbitrary",),
        ),
    )(adj, x, w_p, b_p)
